# Initial kernel scaffold; baseline (speedup 1.0000x reference)
import jax
import jax.numpy as jnp
from jax.experimental import pallas as pl


def kernel(x1, x2, edge_index_v1, edge_index_v2, W_gcn, b_gcn, g_enc, beta_enc, W_proj, b_proj, g_proj, beta_proj, W_pred, b_pred, g_pred, beta_pred):
    raise NotImplementedError("write your pallas kernel here")


# speedup vs baseline: 26.0977x; 26.0977x over previous
import functools

import jax
import jax.numpy as jnp
import numpy as np
from jax import lax
from jax.experimental import pallas as pl
from jax.experimental.pallas import tpu as pltpu
from jax.experimental.pallas import tpu_sc as plsc

N = 10000
D = 128
E = 320000

NP = 10240
ROWS_PT = NP // 16
EPC = 2560
EP = EPC * 128
TCH = EPC // 16
STG = 40
C0 = float(1.0 / np.sqrt(1.0 + 1e-5))

def _deg_body(dst_hbm, deg_hbm, idx_v, ones_v, zbuf_v, hist_sh):
    c = lax.axis_index("c")
    s = lax.axis_index("s")

    def fill_ones(i, _):
        ones_v[pl.ds(i * 16, 16)] = jnp.ones((16,), jnp.float32)
        return 0

    lax.fori_loop(0, 128 // 16, fill_ones, 0)

    def fill_zero(i, _):
        zbuf_v[pl.ds(i * 16, 16)] = jnp.zeros((16,), jnp.float32)
        return 0

    lax.fori_loop(0, ROWS_PT // 16, fill_zero, 0)
    pltpu.sync_copy(zbuf_v, hist_sh.at[pl.ds(s * ROWS_PT, ROWS_PT)])
    pltpu.sync_copy(dst_hbm.at[c, pl.ds(s * TCH, TCH)], idx_v)
    plsc.subcore_barrier()

    def body(j, _):
        pltpu.sync_copy(ones_v, hist_sh.at[idx_v.at[j]], add=True)
        return 0

    lax.fori_loop(0, TCH, body, 0)
    plsc.subcore_barrier()
    pltpu.sync_copy(hist_sh.at[pl.ds(s * ROWS_PT, ROWS_PT)],
                    deg_hbm.at[pl.ds(c * NP + s * ROWS_PT, ROWS_PT)])


def _agg_body(hs_hbm, src_hbm, dst_hbm, agg_hbm, sidx_v, didx_v, rows_v,
              agg_sh, sem):
    c = lax.axis_index("c")
    s = lax.axis_index("s")
    pltpu.sync_copy(hs_hbm.at[pl.ds(c * NP + s * ROWS_PT, ROWS_PT)],
                    agg_sh.at[pl.ds(s * ROWS_PT, ROWS_PT)])
    plsc.subcore_barrier()

    def outer(g, _):
        base = s * TCH + g * STG
        pltpu.sync_copy(src_hbm.at[c, pl.ds(base, STG)], sidx_v)
        pltpu.sync_copy(dst_hbm.at[c, pl.ds(base, STG)], didx_v)

        def body(j, _):
            pltpu.async_copy(hs_hbm.at[sidx_v.at[j]], rows_v, sem).wait()
            pltpu.sync_copy(rows_v, agg_sh.at[didx_v.at[j]], add=True)
            return 0

        lax.fori_loop(0, STG, body, 0)
        return 0

    lax.fori_loop(0, TCH // STG, outer, 0)
    plsc.subcore_barrier()
    pltpu.sync_copy(agg_sh.at[pl.ds(s * ROWS_PT, ROWS_PT)],
                    agg_hbm.at[pl.ds(c * NP + s * ROWS_PT, ROWS_PT)])


@functools.cache
def _sc_kernels():
    mesh = plsc.VectorSubcoreMesh(core_axis_name="c", subcore_axis_name="s")
    deg_sc = pl.kernel(
        _deg_body,
        out_type=jax.ShapeDtypeStruct((2 * NP,), jnp.float32),
        mesh=mesh,
        scratch_types=[
            pltpu.VMEM((TCH, 128), jnp.int32),
            pltpu.VMEM((128,), jnp.float32),
            pltpu.VMEM((ROWS_PT,), jnp.float32),
            pltpu.VMEM_SHARED((NP,), jnp.float32),
        ],
    )
    agg_sc = pl.kernel(
        _agg_body,
        out_type=jax.ShapeDtypeStruct((2 * NP, D), jnp.float32),
        mesh=mesh,
        scratch_types=[
            pltpu.VMEM((STG, 128), jnp.int32),
            pltpu.VMEM((STG, 128), jnp.int32),
            pltpu.VMEM((128, D), jnp.float32),
            pltpu.VMEM_SHARED((NP, D), jnp.float32),
            pltpu.SemaphoreType.DMA,
        ],
    )
    return deg_sc, agg_sc


_HB = 1024


def _hs_body(x_ref, w_ref, deg_ref, hs_ref):
    h = jnp.dot(x_ref[0], w_ref[...], preferred_element_type=jnp.float32)
    dinv = lax.rsqrt(deg_ref[0] + 1.0)
    hs_ref[0] = h * dinv


_hs_tc = pl.pallas_call(
    _hs_body,
    grid=(2, NP // _HB),
    in_specs=[
        pl.BlockSpec((1, _HB, D), lambda v, i: (v, i, 0)),
        pl.BlockSpec((D, D), lambda v, i: (0, 0)),
        pl.BlockSpec((1, _HB, 1), lambda v, i: (v, i, 0)),
    ],
    out_specs=pl.BlockSpec((1, _HB, D), lambda v, i: (v, i, 0)),
    out_shape=jax.ShapeDtypeStruct((2, NP, D), jnp.float32),
)


_FB = 512


def _head_body(agg1_ref, agg2_ref, deg1_ref, deg2_ref, bg_ref, se_ref, be_ref,
               wp_ref, bp_ref, sp_ref, bep_ref, wq_ref, bq_ref, sq_ref,
               beq_ref, rep1_ref, rep2_ref, loss_ref):
    i = pl.program_id(0)

    def view(agg, deg):
        dinv = lax.rsqrt(deg + 1.0)
        rep = agg * dinv + bg_ref[...]
        z = rep * se_ref[...] + be_ref[...]
        proj = jnp.dot(z, wp_ref[...], preferred_element_type=jnp.float32)
        proj = jnp.maximum((proj + bp_ref[...]) * sp_ref[...] + bep_ref[...],
                           0.0)
        prd = jnp.dot(proj, wq_ref[...], preferred_element_type=jnp.float32)
        prd = jnp.maximum((prd + bq_ref[...]) * sq_ref[...] + beq_ref[...],
                          0.0)
        return rep, proj, prd

    rep1, proj1, prd1 = view(agg1_ref[0], deg1_ref[0])
    rep2, proj2, prd2 = view(agg2_ref[0], deg2_ref[0])
    rep1_ref[...] = rep1
    rep2_ref[...] = rep2

    def nrm(x):
        n = jnp.sqrt(jnp.sum(x * x, axis=-1, keepdims=True))
        return x / jnp.maximum(n, 1e-12)

    cos = (jnp.sum(nrm(prd1) * nrm(proj2), axis=-1, keepdims=True) +
           jnp.sum(nrm(prd2) * nrm(proj1), axis=-1, keepdims=True))
    rowid = i * _FB + lax.broadcasted_iota(jnp.int32, (_FB, 1), 0)
    psum = jnp.sum(jnp.where(rowid < N, 4.0 - 2.0 * cos, 0.0))

    @pl.when(i == 0)
    def _():
        loss_ref[...] = jnp.zeros((1, 1), jnp.float32)

    loss_ref[...] += psum


_vec = lambda: pl.BlockSpec((1, D), lambda i: (0, 0))
_head_tc = pl.pallas_call(
    _head_body,
    grid=(NP // _FB,),
    in_specs=[
        pl.BlockSpec((1, _FB, D), lambda i: (0, i, 0)),
        pl.BlockSpec((1, _FB, D), lambda i: (1, i, 0)),
        pl.BlockSpec((1, _FB, 1), lambda i: (0, i, 0)),
        pl.BlockSpec((1, _FB, 1), lambda i: (1, i, 0)),
        _vec(),
        _vec(),
        _vec(),
        pl.BlockSpec((D, D), lambda i: (0, 0)),
        _vec(),
        _vec(),
        _vec(),
        pl.BlockSpec((D, D), lambda i: (0, 0)),
        _vec(),
        _vec(),
        _vec(),
    ],
    out_specs=[
        pl.BlockSpec((_FB, D), lambda i: (i, 0)),
        pl.BlockSpec((_FB, D), lambda i: (i, 0)),
        pl.BlockSpec((1, 1), lambda i: (0, 0)),
    ],
    out_shape=[
        jax.ShapeDtypeStruct((NP, D), jnp.float32),
        jax.ShapeDtypeStruct((NP, D), jnp.float32),
        jax.ShapeDtypeStruct((1, 1), jnp.float32),
    ],
)


def kernel(x1, x2, edge_index_v1, edge_index_v2, W_gcn, b_gcn, g_enc,
           beta_enc, W_proj, b_proj, g_proj, beta_proj, W_pred, b_pred,
           g_pred, beta_pred):
    x = jnp.stack([x1, x2])
    x = jnp.pad(x, ((0, 0), (0, NP - N), (0, 0)))
    pad_idx = jnp.arange(EP - E, dtype=jnp.int32) % (NP - N) + N
    s1 = jnp.concatenate([edge_index_v1[0].astype(jnp.int32), pad_idx])
    t1 = jnp.concatenate([edge_index_v1[1].astype(jnp.int32), pad_idx])
    s2 = jnp.concatenate([edge_index_v2[0].astype(jnp.int32), pad_idx])
    t2 = jnp.concatenate([edge_index_v2[1].astype(jnp.int32), pad_idx])
    src3 = jnp.stack([s1, s2 + NP]).reshape(2, EPC, 128)
    dst3 = jnp.stack([t1, t2]).reshape(2, EPC, 128)

    deg_sc, agg_sc = _sc_kernels()
    deg = deg_sc(dst3)
    deg3 = deg.reshape(2, NP, 1)
    hs = _hs_tc(x, W_gcn.T, deg3)
    agg = agg_sc(hs.reshape(2 * NP, D), src3, dst3)
    agg = agg.reshape(2, NP, D)

    r = lambda v: v.reshape(1, D)
    rep1, rep2, loss_acc = _head_tc(
        agg, agg, deg3, deg3, r(b_gcn), r(g_enc * C0), r(beta_enc), W_proj.T,
        r(b_proj), r(g_proj * C0), r(beta_proj), W_pred.T, r(b_pred),
        r(g_pred * C0), r(beta_pred))
    loss = loss_acc[0, 0] / np.float32(N)
    return rep1[:N], rep2[:N], loss


# speedup vs baseline: 36.1203x; 1.3840x over previous
import functools

import jax
import jax.numpy as jnp
import numpy as np
from jax import lax
from jax.experimental import pallas as pl
from jax.experimental.pallas import tpu as pltpu
from jax.experimental.pallas import tpu_sc as plsc

N = 10000
D = 128
E = 320000

NP = 10240
ROWS_PT = NP // 16
EPC = 2560
EP = EPC * 128
TCH = EPC // 16
STG = 40
C0 = float(1.0 / np.sqrt(1.0 + 1e-5))

def _deg_body(dst_hbm, deg_hbm, idx_v, ones_v, zbuf_v, hist_sh):
    c = lax.axis_index("c")
    s = lax.axis_index("s")

    def fill_ones(i, _):
        ones_v[pl.ds(i * 16, 16)] = jnp.ones((16,), jnp.float32)
        return 0

    lax.fori_loop(0, 128 // 16, fill_ones, 0)

    def fill_zero(i, _):
        zbuf_v[pl.ds(i * 16, 16)] = jnp.zeros((16,), jnp.float32)
        return 0

    lax.fori_loop(0, ROWS_PT // 16, fill_zero, 0)
    pltpu.sync_copy(zbuf_v, hist_sh.at[pl.ds(s * ROWS_PT, ROWS_PT)])
    pltpu.sync_copy(dst_hbm.at[c, pl.ds(s * TCH, TCH)], idx_v)
    plsc.subcore_barrier()

    def body(j, _):
        pltpu.sync_copy(ones_v, hist_sh.at[idx_v.at[j]], add=True)
        return 0

    lax.fori_loop(0, TCH, body, 0)
    plsc.subcore_barrier()
    pltpu.sync_copy(hist_sh.at[pl.ds(s * ROWS_PT, ROWS_PT)],
                    deg_hbm.at[pl.ds(c * NP + s * ROWS_PT, ROWS_PT)])


def _agg_body(hs_hbm, src_hbm, dst_hbm, agg_hbm, sidx_v, didx_v, rows0_v,
              rows1_v, agg_sh, sem0, sem1):
    c = lax.axis_index("c")
    s = lax.axis_index("s")
    pltpu.sync_copy(hs_hbm.at[pl.ds(c * NP + s * ROWS_PT, ROWS_PT)],
                    agg_sh.at[pl.ds(s * ROWS_PT, ROWS_PT)])
    plsc.subcore_barrier()

    def outer(g, _):
        base = s * TCH + g * STG
        pltpu.sync_copy(src_hbm.at[c, pl.ds(base, STG)], sidx_v)
        pltpu.sync_copy(dst_hbm.at[c, pl.ds(base, STG)], didx_v)
        pltpu.async_copy(hs_hbm.at[sidx_v.at[0]], rows0_v, sem0)

        def inner(jj, _):
            a = 2 * jj
            pltpu.async_copy(hs_hbm.at[sidx_v.at[a + 1]], rows1_v, sem1)
            pltpu.make_async_copy(hs_hbm.at[pl.ds(0, 128)], rows0_v,
                                  sem0).wait()
            pltpu.sync_copy(rows0_v, agg_sh.at[didx_v.at[a]], add=True)

            @pl.when(a + 2 < STG)
            def _():
                pltpu.async_copy(hs_hbm.at[sidx_v.at[a + 2]], rows0_v, sem0)

            pltpu.make_async_copy(hs_hbm.at[pl.ds(0, 128)], rows1_v,
                                  sem1).wait()
            pltpu.sync_copy(rows1_v, agg_sh.at[didx_v.at[a + 1]], add=True)
            return 0

        lax.fori_loop(0, STG // 2, inner, 0)
        return 0

    lax.fori_loop(0, TCH // STG, outer, 0)
    plsc.subcore_barrier()
    pltpu.sync_copy(agg_sh.at[pl.ds(s * ROWS_PT, ROWS_PT)],
                    agg_hbm.at[pl.ds(c * NP + s * ROWS_PT, ROWS_PT)])


@functools.cache
def _sc_kernels():
    mesh = plsc.VectorSubcoreMesh(core_axis_name="c", subcore_axis_name="s")
    deg_sc = pl.kernel(
        _deg_body,
        out_type=jax.ShapeDtypeStruct((2 * NP,), jnp.float32),
        mesh=mesh,
        scratch_types=[
            pltpu.VMEM((TCH, 128), jnp.int32),
            pltpu.VMEM((128,), jnp.float32),
            pltpu.VMEM((ROWS_PT,), jnp.float32),
            pltpu.VMEM_SHARED((NP,), jnp.float32),
        ],
    )
    agg_sc = pl.kernel(
        _agg_body,
        out_type=jax.ShapeDtypeStruct((2 * NP, D), jnp.float32),
        mesh=mesh,
        scratch_types=[
            pltpu.VMEM((STG, 128), jnp.int32),
            pltpu.VMEM((STG, 128), jnp.int32),
            pltpu.VMEM((128, D), jnp.float32),
            pltpu.VMEM((128, D), jnp.float32),
            pltpu.VMEM_SHARED((NP, D), jnp.float32),
            pltpu.SemaphoreType.DMA,
            pltpu.SemaphoreType.DMA,
        ],
    )
    return deg_sc, agg_sc


_HB = 1024


def _hs_body(x_ref, w_ref, deg_ref, hs_ref):
    h = jnp.dot(x_ref[0], w_ref[...], preferred_element_type=jnp.float32)
    dinv = lax.rsqrt(deg_ref[0] + 1.0)
    hs_ref[0] = h * dinv


_hs_tc = pl.pallas_call(
    _hs_body,
    grid=(2, NP // _HB),
    in_specs=[
        pl.BlockSpec((1, _HB, D), lambda v, i: (v, i, 0)),
        pl.BlockSpec((D, D), lambda v, i: (0, 0)),
        pl.BlockSpec((1, _HB, 1), lambda v, i: (v, i, 0)),
    ],
    out_specs=pl.BlockSpec((1, _HB, D), lambda v, i: (v, i, 0)),
    out_shape=jax.ShapeDtypeStruct((2, NP, D), jnp.float32),
)


_FB = 512


def _head_body(agg1_ref, agg2_ref, deg1_ref, deg2_ref, bg_ref, se_ref, be_ref,
               wp_ref, bp_ref, sp_ref, bep_ref, wq_ref, bq_ref, sq_ref,
               beq_ref, rep1_ref, rep2_ref, loss_ref):
    i = pl.program_id(0)

    def view(agg, deg):
        dinv = lax.rsqrt(deg + 1.0)
        rep = agg * dinv + bg_ref[...]
        z = rep * se_ref[...] + be_ref[...]
        proj = jnp.dot(z, wp_ref[...], preferred_element_type=jnp.float32)
        proj = jnp.maximum((proj + bp_ref[...]) * sp_ref[...] + bep_ref[...],
                           0.0)
        prd = jnp.dot(proj, wq_ref[...], preferred_element_type=jnp.float32)
        prd = jnp.maximum((prd + bq_ref[...]) * sq_ref[...] + beq_ref[...],
                          0.0)
        return rep, proj, prd

    rep1, proj1, prd1 = view(agg1_ref[0], deg1_ref[0])
    rep2, proj2, prd2 = view(agg2_ref[0], deg2_ref[0])
    rep1_ref[...] = rep1
    rep2_ref[...] = rep2

    def nrm(x):
        n = jnp.sqrt(jnp.sum(x * x, axis=-1, keepdims=True))
        return x / jnp.maximum(n, 1e-12)

    cos = (jnp.sum(nrm(prd1) * nrm(proj2), axis=-1, keepdims=True) +
           jnp.sum(nrm(prd2) * nrm(proj1), axis=-1, keepdims=True))
    rowid = i * _FB + lax.broadcasted_iota(jnp.int32, (_FB, 1), 0)
    psum = jnp.sum(jnp.where(rowid < N, 4.0 - 2.0 * cos, 0.0))

    @pl.when(i == 0)
    def _():
        loss_ref[...] = jnp.zeros((1, 1), jnp.float32)

    loss_ref[...] += psum


_vec = lambda: pl.BlockSpec((1, D), lambda i: (0, 0))
_head_tc = pl.pallas_call(
    _head_body,
    grid=(NP // _FB,),
    in_specs=[
        pl.BlockSpec((1, _FB, D), lambda i: (0, i, 0)),
        pl.BlockSpec((1, _FB, D), lambda i: (1, i, 0)),
        pl.BlockSpec((1, _FB, 1), lambda i: (0, i, 0)),
        pl.BlockSpec((1, _FB, 1), lambda i: (1, i, 0)),
        _vec(),
        _vec(),
        _vec(),
        pl.BlockSpec((D, D), lambda i: (0, 0)),
        _vec(),
        _vec(),
        _vec(),
        pl.BlockSpec((D, D), lambda i: (0, 0)),
        _vec(),
        _vec(),
        _vec(),
    ],
    out_specs=[
        pl.BlockSpec((_FB, D), lambda i: (i, 0)),
        pl.BlockSpec((_FB, D), lambda i: (i, 0)),
        pl.BlockSpec((1, 1), lambda i: (0, 0)),
    ],
    out_shape=[
        jax.ShapeDtypeStruct((NP, D), jnp.float32),
        jax.ShapeDtypeStruct((NP, D), jnp.float32),
        jax.ShapeDtypeStruct((1, 1), jnp.float32),
    ],
)


def kernel(x1, x2, edge_index_v1, edge_index_v2, W_gcn, b_gcn, g_enc,
           beta_enc, W_proj, b_proj, g_proj, beta_proj, W_pred, b_pred,
           g_pred, beta_pred):
    x = jnp.stack([x1, x2])
    x = jnp.pad(x, ((0, 0), (0, NP - N), (0, 0)))
    pad_idx = jnp.arange(EP - E, dtype=jnp.int32) % (NP - N) + N
    s1 = jnp.concatenate([edge_index_v1[0].astype(jnp.int32), pad_idx])
    t1 = jnp.concatenate([edge_index_v1[1].astype(jnp.int32), pad_idx])
    s2 = jnp.concatenate([edge_index_v2[0].astype(jnp.int32), pad_idx])
    t2 = jnp.concatenate([edge_index_v2[1].astype(jnp.int32), pad_idx])
    src3 = jnp.stack([s1, s2 + NP]).reshape(2, EPC, 128)
    dst3 = jnp.stack([t1, t2]).reshape(2, EPC, 128)

    deg_sc, agg_sc = _sc_kernels()
    deg = deg_sc(dst3)
    deg3 = deg.reshape(2, NP, 1)
    hs = _hs_tc(x, W_gcn.T, deg3)
    agg = agg_sc(hs.reshape(2 * NP, D), src3, dst3)
    agg = agg.reshape(2, NP, D)

    r = lambda v: v.reshape(1, D)
    rep1, rep2, loss_acc = _head_tc(
        agg, agg, deg3, deg3, r(b_gcn), r(g_enc * C0), r(beta_enc), W_proj.T,
        r(b_proj), r(g_proj * C0), r(beta_proj), W_pred.T, r(b_pred),
        r(g_pred * C0), r(beta_pred))
    loss = loss_acc[0, 0] / np.float32(N)
    return rep1[:N], rep2[:N], loss


# speedup vs baseline: 39.9168x; 1.1051x over previous
import functools

import jax
import jax.numpy as jnp
import numpy as np
from jax import lax
from jax.experimental import pallas as pl
from jax.experimental.pallas import tpu as pltpu
from jax.experimental.pallas import tpu_sc as plsc

N = 10000
D = 128
E = 320000

NP = 10240
ROWS_PT = NP // 16
EPC = 2560
EP = EPC * 128
TCH = EPC // 16
STG = 40
C0 = float(1.0 / np.sqrt(1.0 + 1e-5))

def _deg_body(dst_hbm, deg_hbm, idx_v, ones_v, zbuf_v, hist_sh):
    c = lax.axis_index("c")
    s = lax.axis_index("s")

    def fill_ones(i, _):
        ones_v[pl.ds(i * 16, 16)] = jnp.ones((16,), jnp.float32)
        return 0

    lax.fori_loop(0, 128 // 16, fill_ones, 0)

    def fill_zero(i, _):
        zbuf_v[pl.ds(i * 16, 16)] = jnp.zeros((16,), jnp.float32)
        return 0

    lax.fori_loop(0, ROWS_PT // 16, fill_zero, 0)
    pltpu.sync_copy(zbuf_v, hist_sh.at[pl.ds(s * ROWS_PT, ROWS_PT)])
    pltpu.sync_copy(dst_hbm.at[c, pl.ds(s * TCH, TCH)], idx_v)
    plsc.subcore_barrier()

    def body(j, _):
        pltpu.sync_copy(ones_v, hist_sh.at[idx_v.at[j]], add=True)
        return 0

    lax.fori_loop(0, TCH, body, 0)
    plsc.subcore_barrier()
    pltpu.sync_copy(hist_sh.at[pl.ds(s * ROWS_PT, ROWS_PT)],
                    deg_hbm.at[pl.ds(c * NP + s * ROWS_PT, ROWS_PT)])


def _agg_body(hs_hbm, src_hbm, dst_hbm, agg_hbm, sidx_v, didx_v, rows0_v,
              rows1_v, agg_sh, sem0, sem1):
    c = lax.axis_index("c")
    s = lax.axis_index("s")
    @pl.when(s < 15)
    def _():
        pltpu.sync_copy(hs_hbm.at[pl.ds(c * N + s * ROWS_PT, ROWS_PT)],
                        agg_sh.at[pl.ds(s * ROWS_PT, ROWS_PT)])

    @pl.when(s == 15)
    def _():
        pltpu.sync_copy(hs_hbm.at[pl.ds(c * N + 15 * ROWS_PT, N - 15 * ROWS_PT)],
                        agg_sh.at[pl.ds(15 * ROWS_PT, N - 15 * ROWS_PT)])

    plsc.subcore_barrier()

    def outer(g, _):
        base = s * TCH + g * STG
        pltpu.sync_copy(src_hbm.at[c, pl.ds(base, STG)], sidx_v)
        pltpu.sync_copy(dst_hbm.at[c, pl.ds(base, STG)], didx_v)
        pltpu.async_copy(hs_hbm.at[sidx_v.at[0]], rows0_v, sem0)

        def inner(jj, _):
            a = 2 * jj
            pltpu.async_copy(hs_hbm.at[sidx_v.at[a + 1]], rows1_v, sem1)
            pltpu.make_async_copy(hs_hbm.at[pl.ds(0, 128)], rows0_v,
                                  sem0).wait()
            pltpu.sync_copy(rows0_v, agg_sh.at[didx_v.at[a]], add=True)

            @pl.when(a + 2 < STG)
            def _():
                pltpu.async_copy(hs_hbm.at[sidx_v.at[a + 2]], rows0_v, sem0)

            pltpu.make_async_copy(hs_hbm.at[pl.ds(0, 128)], rows1_v,
                                  sem1).wait()
            pltpu.sync_copy(rows1_v, agg_sh.at[didx_v.at[a + 1]], add=True)
            return 0

        lax.fori_loop(0, STG // 2, inner, 0)
        return 0

    lax.fori_loop(0, TCH // STG, outer, 0)
    plsc.subcore_barrier()

    @pl.when(s < 15)
    def _():
        pltpu.sync_copy(agg_sh.at[pl.ds(s * ROWS_PT, ROWS_PT)],
                        agg_hbm.at[pl.ds(c * N + s * ROWS_PT, ROWS_PT)])

    @pl.when(s == 15)
    def _():
        pltpu.sync_copy(agg_sh.at[pl.ds(15 * ROWS_PT, N - 15 * ROWS_PT)],
                        agg_hbm.at[pl.ds(c * N + 15 * ROWS_PT, N - 15 * ROWS_PT)])


@functools.cache
def _sc_kernels():
    mesh = plsc.VectorSubcoreMesh(core_axis_name="c", subcore_axis_name="s")
    deg_sc = pl.kernel(
        _deg_body,
        out_type=jax.ShapeDtypeStruct((2 * NP,), jnp.float32),
        mesh=mesh,
        scratch_types=[
            pltpu.VMEM((TCH, 128), jnp.int32),
            pltpu.VMEM((128,), jnp.float32),
            pltpu.VMEM((ROWS_PT,), jnp.float32),
            pltpu.VMEM_SHARED((NP,), jnp.float32),
        ],
    )
    agg_sc = pl.kernel(
        _agg_body,
        out_type=jax.ShapeDtypeStruct((2 * N, D), jnp.float32),
        mesh=mesh,
        scratch_types=[
            pltpu.VMEM((STG, 128), jnp.int32),
            pltpu.VMEM((STG, 128), jnp.int32),
            pltpu.VMEM((128, D), jnp.float32),
            pltpu.VMEM((128, D), jnp.float32),
            pltpu.VMEM_SHARED((NP, D), jnp.float32),
            pltpu.SemaphoreType.DMA,
            pltpu.SemaphoreType.DMA,
        ],
    )
    return deg_sc, agg_sc


_HB = 2000


def _hs_body(x_ref, w_ref, deg_ref, hs_ref):
    h = jnp.dot(x_ref[0], w_ref[...], preferred_element_type=jnp.float32)
    dinv = lax.rsqrt(deg_ref[0] + 1.0)
    hs_ref[0] = h * dinv


_hs_tc = pl.pallas_call(
    _hs_body,
    grid=(2, N // _HB),
    in_specs=[
        pl.BlockSpec((1, _HB, D), lambda v, i: (v, i, 0)),
        pl.BlockSpec((D, D), lambda v, i: (0, 0)),
        pl.BlockSpec((1, _HB, 1), lambda v, i: (v, i, 0)),
    ],
    out_specs=pl.BlockSpec((1, _HB, D), lambda v, i: (v, i, 0)),
    out_shape=jax.ShapeDtypeStruct((2, N, D), jnp.float32),
)


_FB = 1000


def _head_body(agg1_ref, agg2_ref, deg1_ref, deg2_ref, bg_ref, se_ref, be_ref,
               wp_ref, bp_ref, sp_ref, bep_ref, wq_ref, bq_ref, sq_ref,
               beq_ref, rep1_ref, rep2_ref, loss_ref):
    i = pl.program_id(0)

    def view(agg, deg):
        dinv = lax.rsqrt(deg + 1.0)
        rep = agg * dinv + bg_ref[...]
        z = rep * se_ref[...] + be_ref[...]
        proj = jnp.dot(z, wp_ref[...], preferred_element_type=jnp.float32)
        proj = jnp.maximum((proj + bp_ref[...]) * sp_ref[...] + bep_ref[...],
                           0.0)
        prd = jnp.dot(proj, wq_ref[...], preferred_element_type=jnp.float32)
        prd = jnp.maximum((prd + bq_ref[...]) * sq_ref[...] + beq_ref[...],
                          0.0)
        return rep, proj, prd

    rep1, proj1, prd1 = view(agg1_ref[0], deg1_ref[0])
    rep2, proj2, prd2 = view(agg2_ref[0], deg2_ref[0])
    rep1_ref[...] = rep1
    rep2_ref[...] = rep2

    def nrm(x):
        n = jnp.sqrt(jnp.sum(x * x, axis=-1, keepdims=True))
        return x / jnp.maximum(n, 1e-12)

    cos = (jnp.sum(nrm(prd1) * nrm(proj2), axis=-1, keepdims=True) +
           jnp.sum(nrm(prd2) * nrm(proj1), axis=-1, keepdims=True))
    psum = jnp.sum(4.0 - 2.0 * cos)

    @pl.when(i == 0)
    def _():
        loss_ref[...] = jnp.zeros((1, 1), jnp.float32)

    loss_ref[...] += psum


_vec = lambda: pl.BlockSpec((1, D), lambda i: (0, 0))
_head_tc = pl.pallas_call(
    _head_body,
    grid=(N // _FB,),
    in_specs=[
        pl.BlockSpec((1, _FB, D), lambda i: (0, i, 0)),
        pl.BlockSpec((1, _FB, D), lambda i: (1, i, 0)),
        pl.BlockSpec((1, _FB, 1), lambda i: (0, i, 0)),
        pl.BlockSpec((1, _FB, 1), lambda i: (1, i, 0)),
        _vec(),
        _vec(),
        _vec(),
        pl.BlockSpec((D, D), lambda i: (0, 0)),
        _vec(),
        _vec(),
        _vec(),
        pl.BlockSpec((D, D), lambda i: (0, 0)),
        _vec(),
        _vec(),
        _vec(),
    ],
    out_specs=[
        pl.BlockSpec((_FB, D), lambda i: (i, 0)),
        pl.BlockSpec((_FB, D), lambda i: (i, 0)),
        pl.BlockSpec((1, 1), lambda i: (0, 0)),
    ],
    out_shape=[
        jax.ShapeDtypeStruct((N, D), jnp.float32),
        jax.ShapeDtypeStruct((N, D), jnp.float32),
        jax.ShapeDtypeStruct((1, 1), jnp.float32),
    ],
)


def kernel(x1, x2, edge_index_v1, edge_index_v2, W_gcn, b_gcn, g_enc,
           beta_enc, W_proj, b_proj, g_proj, beta_proj, W_pred, b_pred,
           g_pred, beta_pred):
    x = jnp.stack([x1, x2])
    pad_src = jnp.arange(EP - E, dtype=jnp.int32) % N
    pad_dst = jnp.arange(EP - E, dtype=jnp.int32) % (NP - N) + N
    s1 = jnp.concatenate([edge_index_v1[0].astype(jnp.int32), pad_src])
    t1 = jnp.concatenate([edge_index_v1[1].astype(jnp.int32), pad_dst])
    s2 = jnp.concatenate([edge_index_v2[0].astype(jnp.int32), pad_src])
    t2 = jnp.concatenate([edge_index_v2[1].astype(jnp.int32), pad_dst])
    src3 = jnp.stack([s1, s2 + N]).reshape(2, EPC, 128)
    dst3 = jnp.stack([t1, t2]).reshape(2, EPC, 128)

    deg_sc, agg_sc = _sc_kernels()
    degf = deg_sc(dst3)
    degv = jnp.stack([degf[:N], degf[NP:NP + N]]).reshape(2, N, 1)
    hs = _hs_tc(x, W_gcn.T, degv)
    agg = agg_sc(hs.reshape(2 * N, D), src3, dst3)
    agg = agg.reshape(2, N, D)

    r = lambda v: v.reshape(1, D)
    rep1, rep2, loss_acc = _head_tc(
        agg, agg, degv, degv, r(b_gcn), r(g_enc * C0), r(beta_enc), W_proj.T,
        r(b_proj), r(g_proj * C0), r(beta_proj), W_pred.T, r(b_pred),
        r(g_pred * C0), r(beta_pred))
    loss = loss_acc[0, 0] / np.float32(N)
    return rep1, rep2, loss


# speedup vs baseline: 40.3572x; 1.0110x over previous
import functools

import jax
import jax.numpy as jnp
import numpy as np
from jax import lax
from jax.experimental import pallas as pl
from jax.experimental.pallas import tpu as pltpu
from jax.experimental.pallas import tpu_sc as plsc

N = 10000
D = 128
E = 320000

NP = 10240
ROWS_PT = NP // 16
EPC = 2560
EP = EPC * 128
TCH = EPC // 16
STG = 40
C0 = float(1.0 / np.sqrt(1.0 + 1e-5))

def _deg_body(dst_hbm, deg_hbm, idx_v, histp_v, acc_v, hists_sh):
    c = lax.axis_index("c")
    s = lax.axis_index("s")

    def fz(i, _):
        histp_v[pl.ds(i * 16, 16)] = jnp.zeros((16,), jnp.float32)
        return 0

    lax.fori_loop(0, NP // 16, fz, 0)
    pltpu.sync_copy(dst_hbm.at[c, pl.ds(s * TCH, TCH)], idx_v)
    ones = jnp.ones((16,), jnp.float32)

    def body(j, _):
        def inner(k, _):
            ii = idx_v[j, pl.ds(k * 16, 16)]
            plsc.addupdate_scatter(histp_v, [ii], ones)
            return 0

        lax.fori_loop(0, 128 // 16, inner, 0)
        return 0

    lax.fori_loop(0, TCH, body, 0)
    pltpu.sync_copy(histp_v, hists_sh.at[s])
    plsc.subcore_barrier()
    for r in range(16):
        pltpu.sync_copy(hists_sh.at[r, pl.ds(s * ROWS_PT, ROWS_PT)],
                        acc_v.at[r])

    def red(k, _):
        tot = acc_v[0, pl.ds(k * 16, 16)]
        for r in range(1, 16):
            tot = tot + acc_v[r, pl.ds(k * 16, 16)]
        histp_v[pl.ds(k * 16, 16)] = tot
        return 0

    lax.fori_loop(0, ROWS_PT // 16, red, 0)
    pltpu.sync_copy(histp_v.at[pl.ds(0, ROWS_PT)],
                    deg_hbm.at[pl.ds(c * NP + s * ROWS_PT, ROWS_PT)])


def _agg_body(hs_hbm, src_hbm, dst_hbm, agg_hbm, sidx_v, didx_v, rows0_v,
              rows1_v, agg_sh, sem0, sem1):
    c = lax.axis_index("c")
    s = lax.axis_index("s")
    @pl.when(s < 15)
    def _():
        pltpu.sync_copy(hs_hbm.at[pl.ds(c * N + s * ROWS_PT, ROWS_PT)],
                        agg_sh.at[pl.ds(s * ROWS_PT, ROWS_PT)])

    @pl.when(s == 15)
    def _():
        pltpu.sync_copy(hs_hbm.at[pl.ds(c * N + 15 * ROWS_PT, N - 15 * ROWS_PT)],
                        agg_sh.at[pl.ds(15 * ROWS_PT, N - 15 * ROWS_PT)])

    plsc.subcore_barrier()

    def outer(g, _):
        base = s * TCH + g * STG
        pltpu.sync_copy(src_hbm.at[c, pl.ds(base, STG)], sidx_v)
        pltpu.sync_copy(dst_hbm.at[c, pl.ds(base, STG)], didx_v)
        pltpu.async_copy(hs_hbm.at[sidx_v.at[0]], rows0_v, sem0)

        def inner(jj, _):
            a = 2 * jj
            pltpu.async_copy(hs_hbm.at[sidx_v.at[a + 1]], rows1_v, sem1)
            pltpu.make_async_copy(hs_hbm.at[pl.ds(0, 128)], rows0_v,
                                  sem0).wait()
            pltpu.sync_copy(rows0_v, agg_sh.at[didx_v.at[a]], add=True)

            @pl.when(a + 2 < STG)
            def _():
                pltpu.async_copy(hs_hbm.at[sidx_v.at[a + 2]], rows0_v, sem0)

            pltpu.make_async_copy(hs_hbm.at[pl.ds(0, 128)], rows1_v,
                                  sem1).wait()
            pltpu.sync_copy(rows1_v, agg_sh.at[didx_v.at[a + 1]], add=True)
            return 0

        lax.fori_loop(0, STG // 2, inner, 0)
        return 0

    lax.fori_loop(0, TCH // STG, outer, 0)
    plsc.subcore_barrier()

    @pl.when(s < 15)
    def _():
        pltpu.sync_copy(agg_sh.at[pl.ds(s * ROWS_PT, ROWS_PT)],
                        agg_hbm.at[pl.ds(c * N + s * ROWS_PT, ROWS_PT)])

    @pl.when(s == 15)
    def _():
        pltpu.sync_copy(agg_sh.at[pl.ds(15 * ROWS_PT, N - 15 * ROWS_PT)],
                        agg_hbm.at[pl.ds(c * N + 15 * ROWS_PT, N - 15 * ROWS_PT)])


@functools.cache
def _sc_kernels():
    mesh = plsc.VectorSubcoreMesh(core_axis_name="c", subcore_axis_name="s")
    deg_sc = pl.kernel(
        _deg_body,
        out_type=jax.ShapeDtypeStruct((2 * NP,), jnp.float32),
        mesh=mesh,
        compiler_params=pltpu.CompilerParams(needs_layout_passes=False),
        scratch_types=[
            pltpu.VMEM((TCH, 128), jnp.int32),
            pltpu.VMEM((NP,), jnp.float32),
            pltpu.VMEM((16, ROWS_PT), jnp.float32),
            pltpu.VMEM_SHARED((16, NP), jnp.float32),
        ],
    )
    agg_sc = pl.kernel(
        _agg_body,
        out_type=jax.ShapeDtypeStruct((2 * N, D), jnp.float32),
        mesh=mesh,
        scratch_types=[
            pltpu.VMEM((STG, 128), jnp.int32),
            pltpu.VMEM((STG, 128), jnp.int32),
            pltpu.VMEM((128, D), jnp.float32),
            pltpu.VMEM((128, D), jnp.float32),
            pltpu.VMEM_SHARED((NP, D), jnp.float32),
            pltpu.SemaphoreType.DMA,
            pltpu.SemaphoreType.DMA,
        ],
    )
    return deg_sc, agg_sc


_HB = 2000


def _hs_body(x_ref, w_ref, deg_ref, hs_ref):
    h = jnp.dot(x_ref[0], w_ref[...], preferred_element_type=jnp.float32)
    dinv = lax.rsqrt(deg_ref[0] + 1.0)
    hs_ref[0] = h * dinv


_hs_tc = pl.pallas_call(
    _hs_body,
    grid=(2, N // _HB),
    in_specs=[
        pl.BlockSpec((1, _HB, D), lambda v, i: (v, i, 0)),
        pl.BlockSpec((D, D), lambda v, i: (0, 0)),
        pl.BlockSpec((1, _HB, 1), lambda v, i: (v, i, 0)),
    ],
    out_specs=pl.BlockSpec((1, _HB, D), lambda v, i: (v, i, 0)),
    out_shape=jax.ShapeDtypeStruct((2, N, D), jnp.float32),
)


_FB = 1000


def _head_body(agg1_ref, agg2_ref, deg1_ref, deg2_ref, bg_ref, se_ref, be_ref,
               wp_ref, bp_ref, sp_ref, bep_ref, wq_ref, bq_ref, sq_ref,
               beq_ref, rep1_ref, rep2_ref, loss_ref):
    i = pl.program_id(0)

    def view(agg, deg):
        dinv = lax.rsqrt(deg + 1.0)
        rep = agg * dinv + bg_ref[...]
        z = rep * se_ref[...] + be_ref[...]
        proj = jnp.dot(z, wp_ref[...], preferred_element_type=jnp.float32)
        proj = jnp.maximum((proj + bp_ref[...]) * sp_ref[...] + bep_ref[...],
                           0.0)
        prd = jnp.dot(proj, wq_ref[...], preferred_element_type=jnp.float32)
        prd = jnp.maximum((prd + bq_ref[...]) * sq_ref[...] + beq_ref[...],
                          0.0)
        return rep, proj, prd

    rep1, proj1, prd1 = view(agg1_ref[0], deg1_ref[0])
    rep2, proj2, prd2 = view(agg2_ref[0], deg2_ref[0])
    rep1_ref[...] = rep1
    rep2_ref[...] = rep2

    def nrm(x):
        n = jnp.sqrt(jnp.sum(x * x, axis=-1, keepdims=True))
        return x / jnp.maximum(n, 1e-12)

    cos = (jnp.sum(nrm(prd1) * nrm(proj2), axis=-1, keepdims=True) +
           jnp.sum(nrm(prd2) * nrm(proj1), axis=-1, keepdims=True))
    psum = jnp.sum(4.0 - 2.0 * cos)

    @pl.when(i == 0)
    def _():
        loss_ref[...] = jnp.zeros((1, 1), jnp.float32)

    loss_ref[...] += psum


_vec = lambda: pl.BlockSpec((1, D), lambda i: (0, 0))
_head_tc = pl.pallas_call(
    _head_body,
    grid=(N // _FB,),
    in_specs=[
        pl.BlockSpec((1, _FB, D), lambda i: (0, i, 0)),
        pl.BlockSpec((1, _FB, D), lambda i: (1, i, 0)),
        pl.BlockSpec((1, _FB, 1), lambda i: (0, i, 0)),
        pl.BlockSpec((1, _FB, 1), lambda i: (1, i, 0)),
        _vec(),
        _vec(),
        _vec(),
        pl.BlockSpec((D, D), lambda i: (0, 0)),
        _vec(),
        _vec(),
        _vec(),
        pl.BlockSpec((D, D), lambda i: (0, 0)),
        _vec(),
        _vec(),
        _vec(),
    ],
    out_specs=[
        pl.BlockSpec((_FB, D), lambda i: (i, 0)),
        pl.BlockSpec((_FB, D), lambda i: (i, 0)),
        pl.BlockSpec((1, 1), lambda i: (0, 0)),
    ],
    out_shape=[
        jax.ShapeDtypeStruct((N, D), jnp.float32),
        jax.ShapeDtypeStruct((N, D), jnp.float32),
        jax.ShapeDtypeStruct((1, 1), jnp.float32),
    ],
)


def kernel(x1, x2, edge_index_v1, edge_index_v2, W_gcn, b_gcn, g_enc,
           beta_enc, W_proj, b_proj, g_proj, beta_proj, W_pred, b_pred,
           g_pred, beta_pred):
    x = jnp.stack([x1, x2])
    pad_src = jnp.arange(EP - E, dtype=jnp.int32) % N
    pad_dst = jnp.arange(EP - E, dtype=jnp.int32) % (NP - N) + N
    s1 = jnp.concatenate([edge_index_v1[0].astype(jnp.int32), pad_src])
    t1 = jnp.concatenate([edge_index_v1[1].astype(jnp.int32), pad_dst])
    s2 = jnp.concatenate([edge_index_v2[0].astype(jnp.int32), pad_src])
    t2 = jnp.concatenate([edge_index_v2[1].astype(jnp.int32), pad_dst])
    src3 = jnp.stack([s1, s2 + N]).reshape(2, EPC, 128)
    dst3 = jnp.stack([t1, t2]).reshape(2, EPC, 128)

    deg_sc, agg_sc = _sc_kernels()
    degf = deg_sc(dst3)
    degv = jnp.stack([degf[:N], degf[NP:NP + N]]).reshape(2, N, 1)
    hs = _hs_tc(x, W_gcn.T, degv)
    agg = agg_sc(hs.reshape(2 * N, D), src3, dst3)
    agg = agg.reshape(2, N, D)

    r = lambda v: v.reshape(1, D)
    rep1, rep2, loss_acc = _head_tc(
        agg, agg, degv, degv, r(b_gcn), r(g_enc * C0), r(beta_enc), W_proj.T,
        r(b_proj), r(g_proj * C0), r(beta_proj), W_pred.T, r(b_pred),
        r(g_pred * C0), r(beta_pred))
    loss = loss_acc[0, 0] / np.float32(N)
    return rep1, rep2, loss
